# Initial kernel scaffold; baseline (speedup 1.0000x reference)
import jax
import jax.numpy as jnp
from jax.experimental import pallas as pl


def kernel(z, cls_label, batch):
    raise NotImplementedError("write your pallas kernel here")


# speedup vs baseline: 8.8139x; 8.8139x over previous
import jax
import jax.numpy as jnp
from jax.experimental import pallas as pl

N = 4096
D = 128
TILE = 512


def _seg_kernel(zi_ref, zj_ref, cr_ref, cc_ref, br_ref, bc_ref, out_ref):
    i = pl.program_id(0)
    j = pl.program_id(1)

    br = br_ref[...]
    bc = bc_ref[...]
    r_lo = br[0, 0]
    r_hi = br[TILE - 1, 0]
    c_lo = bc[0, 0]
    c_hi = bc[0, TILE - 1]
    overlap = (r_hi >= c_lo) & (c_hi >= r_lo)

    @pl.when(overlap)
    def _compute():
        gram = jax.lax.dot_general(
            zi_ref[...], zj_ref[...],
            dimension_numbers=(((1,), (1,)), ((), ())),
            preferred_element_type=jnp.float32,
        )
        cr = cr_ref[...]
        cc = cc_ref[...]
        valid_r = ~((cr == 24) | (cr == 25) | (cr == 26))
        valid_c = ~((cc == 24) | (cc == 25) | (cc == 26))
        mask = (cr == cc) & (br == bc) & valid_r & valid_c
        row_ids = jax.lax.broadcasted_iota(jnp.int32, (TILE, TILE), 0) + i * TILE
        col_ids = jax.lax.broadcasted_iota(jnp.int32, (TILE, TILE), 1) + j * TILE
        mask = mask & (row_ids != col_ids)
        out_ref[...] = jnp.where(mask, gram, 0.0)

    @pl.when(~overlap)
    def _zero():
        out_ref[...] = jnp.zeros((TILE, TILE), jnp.float32)


def kernel(z, cls_label, batch):
    cls_col = cls_label.reshape(1, N)
    cls_row = cls_label.reshape(N, 1)
    b_col = batch.reshape(1, N)
    b_row = batch.reshape(N, 1)
    grid = (N // TILE, N // TILE)
    return pl.pallas_call(
        _seg_kernel,
        grid=grid,
        in_specs=[
            pl.BlockSpec((TILE, D), lambda i, j: (i, 0)),
            pl.BlockSpec((TILE, D), lambda i, j: (j, 0)),
            pl.BlockSpec((TILE, 1), lambda i, j: (i, 0)),
            pl.BlockSpec((1, TILE), lambda i, j: (0, j)),
            pl.BlockSpec((TILE, 1), lambda i, j: (i, 0)),
            pl.BlockSpec((1, TILE), lambda i, j: (0, j)),
        ],
        out_specs=pl.BlockSpec((TILE, TILE), lambda i, j: (i, j)),
        out_shape=jax.ShapeDtypeStruct((N, N), jnp.float32),
    )(z, z, cls_row, cls_col, b_row, b_col)


# speedup vs baseline: 9.6850x; 1.0988x over previous
import jax
import jax.numpy as jnp
from jax.experimental import pallas as pl

N = 4096
D = 128
TILE = 512


def _seg_kernel(zi_ref, zj_ref, kr_ref, kc_ref, br_ref, bc_ref, out_ref):
    i = pl.program_id(0)
    j = pl.program_id(1)

    r_lo = br_ref[0, 0]
    r_hi = br_ref[TILE - 1, 0]
    c_lo = bc_ref[0, 0]
    c_hi = bc_ref[0, TILE - 1]
    overlap = (r_hi >= c_lo) & (c_hi >= r_lo)

    @pl.when(overlap)
    def _compute():
        gram = jax.lax.dot_general(
            zi_ref[...], zj_ref[...],
            dimension_numbers=(((1,), (1,)), ((), ())),
            preferred_element_type=jnp.float32,
        )
        mask = kr_ref[...] == kc_ref[...]
        out_ref[...] = jnp.where(mask, gram, 0.0)

    @pl.when(~overlap)
    def _zero():
        out_ref[...] = jnp.zeros((TILE, TILE), jnp.float32)

    @pl.when(i == j)
    def _zero_diag():
        r = jax.lax.broadcasted_iota(jnp.int32, (TILE, TILE), 0)
        c = jax.lax.broadcasted_iota(jnp.int32, (TILE, TILE), 1)
        out_ref[...] = jnp.where(r == c, 0.0, out_ref[...])


def kernel(z, cls_label, batch):
    valid = ~((cls_label == 24) | (cls_label == 25) | (cls_label == 26))
    key = batch * 64 + cls_label
    key_row = jnp.where(valid, key, -1).reshape(N, 1)
    key_col = jnp.where(valid, key, -2).reshape(1, N)
    b_row = batch.reshape(N, 1)
    b_col = batch.reshape(1, N)
    grid = (N // TILE, N // TILE)
    return pl.pallas_call(
        _seg_kernel,
        grid=grid,
        in_specs=[
            pl.BlockSpec((TILE, D), lambda i, j: (i, 0)),
            pl.BlockSpec((TILE, D), lambda i, j: (j, 0)),
            pl.BlockSpec((TILE, 1), lambda i, j: (i, 0)),
            pl.BlockSpec((1, TILE), lambda i, j: (0, j)),
            pl.BlockSpec((TILE, 1), lambda i, j: (i, 0)),
            pl.BlockSpec((1, TILE), lambda i, j: (0, j)),
        ],
        out_specs=pl.BlockSpec((TILE, TILE), lambda i, j: (i, j)),
        out_shape=jax.ShapeDtypeStruct((N, N), jnp.float32),
    )(z, z, key_row, key_col, b_row, b_col)


# speedup vs baseline: 9.9809x; 1.0306x over previous
import jax
import jax.numpy as jnp
from jax.experimental import pallas as pl
from jax.experimental.pallas import tpu as pltpu

N = 4096
D = 128
TILE = 512


def _seg_kernel(s_ref, zi_ref, zj_ref, kr_ref, kc_ref, out_ref):
    i = pl.program_id(0)
    j = pl.program_id(1)

    r_lo = s_ref[0, i]
    r_hi = s_ref[1, i]
    c_lo = s_ref[0, j]
    c_hi = s_ref[1, j]
    overlap = (r_hi >= c_lo) & (c_hi >= r_lo)

    @pl.when(overlap)
    def _compute():
        gram = jax.lax.dot_general(
            zi_ref[...], zj_ref[...],
            dimension_numbers=(((1,), (1,)), ((), ())),
            preferred_element_type=jnp.float32,
        )
        mask = kr_ref[...] == kc_ref[...]
        out_ref[...] = jnp.where(mask, gram, 0.0)

    @pl.when(~overlap)
    def _zero():
        out_ref[...] = jnp.zeros((TILE, TILE), jnp.float32)

    @pl.when(i == j)
    def _zero_diag():
        r = jax.lax.broadcasted_iota(jnp.int32, (TILE, TILE), 0)
        c = jax.lax.broadcasted_iota(jnp.int32, (TILE, TILE), 1)
        out_ref[...] = jnp.where(r == c, 0.0, out_ref[...])


def kernel(z, cls_label, batch):
    valid = ~((cls_label == 24) | (cls_label == 25) | (cls_label == 26))
    key = batch * 64 + cls_label
    key_row = jnp.where(valid, key, -1).reshape(N, 1)
    key_col = jnp.where(valid, key, -2).reshape(1, N)
    tile_lo = batch[::TILE]
    tile_hi = batch[TILE - 1::TILE]
    endpoints = jnp.stack([tile_lo, tile_hi])
    grid = (N // TILE, N // TILE)
    grid_spec = pltpu.PrefetchScalarGridSpec(
        num_scalar_prefetch=1,
        grid=grid,
        in_specs=[
            pl.BlockSpec((TILE, D), lambda i, j, s: (i, 0)),
            pl.BlockSpec((TILE, D), lambda i, j, s: (j, 0)),
            pl.BlockSpec((TILE, 1), lambda i, j, s: (i, 0)),
            pl.BlockSpec((1, TILE), lambda i, j, s: (0, j)),
        ],
        out_specs=pl.BlockSpec((TILE, TILE), lambda i, j, s: (i, j)),
    )
    return pl.pallas_call(
        _seg_kernel,
        grid_spec=grid_spec,
        out_shape=jax.ShapeDtypeStruct((N, N), jnp.float32),
    )(endpoints, z, z, key_row, key_col)


# speedup vs baseline: 17.3029x; 1.7336x over previous
import jax
import jax.numpy as jnp
from jax.experimental import pallas as pl
from jax.experimental.pallas import tpu as pltpu

N = 4096
D = 128
TILE = 1024


def _seg_kernel(s_ref, zi_ref, zj_ref, kr_ref, kc_ref, out_ref):
    i = pl.program_id(0)
    j = pl.program_id(1)

    r_lo = s_ref[0, i]
    r_hi = s_ref[1, i]
    c_lo = s_ref[0, j]
    c_hi = s_ref[1, j]
    overlap = (r_hi >= c_lo) & (c_hi >= r_lo)

    @pl.when(overlap)
    def _compute():
        gram = jax.lax.dot_general(
            zi_ref[...], zj_ref[...],
            dimension_numbers=(((1,), (1,)), ((), ())),
            preferred_element_type=jnp.float32,
        )
        mask = kr_ref[...] == kc_ref[...]
        out_ref[...] = jnp.where(mask, gram, 0.0)

    @pl.when(~overlap)
    def _zero():
        out_ref[...] = jnp.zeros((TILE, TILE), jnp.float32)

    @pl.when(i == j)
    def _zero_diag():
        r = jax.lax.broadcasted_iota(jnp.int32, (TILE, TILE), 0)
        c = jax.lax.broadcasted_iota(jnp.int32, (TILE, TILE), 1)
        out_ref[...] = jnp.where(r == c, 0.0, out_ref[...])


def kernel(z, cls_label, batch):
    valid = ~((cls_label == 24) | (cls_label == 25) | (cls_label == 26))
    key = batch * 64 + cls_label
    key_row = jnp.where(valid, key, -1).reshape(N, 1)
    key_col = jnp.where(valid, key, -2).reshape(1, N)
    tile_lo = batch[::TILE]
    tile_hi = batch[TILE - 1::TILE]
    endpoints = jnp.stack([tile_lo, tile_hi])
    grid = (N // TILE, N // TILE)
    grid_spec = pltpu.PrefetchScalarGridSpec(
        num_scalar_prefetch=1,
        grid=grid,
        in_specs=[
            pl.BlockSpec((TILE, D), lambda i, j, s: (i, 0)),
            pl.BlockSpec((TILE, D), lambda i, j, s: (j, 0)),
            pl.BlockSpec((TILE, 1), lambda i, j, s: (i, 0)),
            pl.BlockSpec((1, TILE), lambda i, j, s: (0, j)),
        ],
        out_specs=pl.BlockSpec((TILE, TILE), lambda i, j, s: (i, j)),
    )
    return pl.pallas_call(
        _seg_kernel,
        grid_spec=grid_spec,
        out_shape=jax.ShapeDtypeStruct((N, N), jnp.float32),
    )(endpoints, z, z, key_row, key_col)


# speedup vs baseline: 18.7686x; 1.0847x over previous
import jax
import jax.numpy as jnp
from jax.experimental import pallas as pl
from jax.experimental.pallas import tpu as pltpu

N = 4096
D = 128
TILE = 2048


def _seg_kernel(s_ref, zi_ref, zj_ref, kr_ref, kc_ref, out_ref):
    i = pl.program_id(0)
    j = pl.program_id(1)

    r_lo = s_ref[0, i]
    r_hi = s_ref[1, i]
    c_lo = s_ref[0, j]
    c_hi = s_ref[1, j]
    overlap = (r_hi >= c_lo) & (c_hi >= r_lo)

    @pl.when(overlap)
    def _compute():
        gram = jax.lax.dot_general(
            zi_ref[...], zj_ref[...],
            dimension_numbers=(((1,), (1,)), ((), ())),
            preferred_element_type=jnp.float32,
        )
        mask = kr_ref[...] == kc_ref[...]
        out_ref[...] = jnp.where(mask, gram, 0.0)

    @pl.when(~overlap)
    def _zero():
        out_ref[...] = jnp.zeros((TILE, TILE), jnp.float32)

    @pl.when(i == j)
    def _zero_diag():
        r = jax.lax.broadcasted_iota(jnp.int32, (TILE, TILE), 0)
        c = jax.lax.broadcasted_iota(jnp.int32, (TILE, TILE), 1)
        out_ref[...] = jnp.where(r == c, 0.0, out_ref[...])


def kernel(z, cls_label, batch):
    valid = ~((cls_label == 24) | (cls_label == 25) | (cls_label == 26))
    key = batch * 64 + cls_label
    key_row = jnp.where(valid, key, -1).reshape(N, 1)
    key_col = jnp.where(valid, key, -2).reshape(1, N)
    tile_lo = batch[::TILE]
    tile_hi = batch[TILE - 1::TILE]
    endpoints = jnp.stack([tile_lo, tile_hi])
    grid = (N // TILE, N // TILE)
    grid_spec = pltpu.PrefetchScalarGridSpec(
        num_scalar_prefetch=1,
        grid=grid,
        in_specs=[
            pl.BlockSpec((TILE, D), lambda i, j, s: (i, 0)),
            pl.BlockSpec((TILE, D), lambda i, j, s: (j, 0)),
            pl.BlockSpec((TILE, 1), lambda i, j, s: (i, 0)),
            pl.BlockSpec((1, TILE), lambda i, j, s: (0, j)),
        ],
        out_specs=pl.BlockSpec((TILE, TILE), lambda i, j, s: (i, j)),
    )
    return pl.pallas_call(
        _seg_kernel,
        grid_spec=grid_spec,
        out_shape=jax.ShapeDtypeStruct((N, N), jnp.float32),
    )(endpoints, z, z, key_row, key_col)
